# baseline (speedup 1.0000x reference)
import jax
import jax.numpy as jnp
from jax import lax
from jax.experimental import pallas as pl
from jax.experimental.pallas import tpu as pltpu
from jax.experimental.pallas import tpu_sc as plsc

BATCH, MAXLEN, EMBED = 4, 2048, 1024
NC, NS = 2, 16
NW = NC * NS
T_PER_W = MAXLEN // NW
ROWS_SUB = 16
WORDS_SUB = ROWS_SUB * EMBED
N_SUB = (T_PER_W // ROWS_SUB) * BATCH


def _sc_body(x_hbm, pos_hbm, out_hbm, pos_v, xb0, xb1, xb2,
             si0, si1, si2, so0, so1, so2):
    wid = lax.axis_index("s") * NC + lax.axis_index("c")
    t0 = wid * T_PER_W
    pltpu.sync_copy(pos_hbm.at[pl.ds(t0 * EMBED, T_PER_W * EMBED)], pos_v)

    bufs = (xb0, xb1, xb2)
    isems = (si0, si1, si2)
    osems = (so0, so1, so2)
    n_rc = T_PER_W // ROWS_SUB

    def xoff(step):
        b, rc = divmod(step, n_rc)
        return (b * MAXLEN + t0 + rc * ROWS_SUB) * EMBED

    def start_in(step):
        p = step % 3
        return pltpu.async_copy(
            x_hbm.at[pl.ds(xoff(step), WORDS_SUB)], bufs[p], isems[p])

    def start_out(step):
        p = step % 3
        return pltpu.async_copy(
            bufs[p], out_hbm.at[pl.ds(xoff(step), WORDS_SUB)], osems[p])

    h_in = [start_in(0), start_in(1), start_in(2)]
    h_out = [None, None, None]
    for step in range(N_SUB):
        p = step % 3
        nxt = step + 1
        if 3 <= nxt < N_SUB:
            q = nxt % 3
            h_out[q].wait()
            h_in[q] = start_in(nxt)
        h_in[p].wait()
        buf = bufs[p]
        pos_base = (step % n_rc) * WORDS_SUB

        @plsc.parallel_loop(0, WORDS_SUB, 16, unroll=8)
        def _(i, _buf=buf, _pb=pos_base):
            _buf[pl.ds(i, 16)] = _buf[pl.ds(i, 16)] + pos_v[pl.ds(_pb + i, 16)]

        h_out[p] = start_out(step)
    for p in range(3):
        h_out[p].wait()


def _sc_add(x, pos_table):
    xf = x.reshape(-1)
    pf = pos_table.reshape(-1)
    out = pl.kernel(
        _sc_body,
        out_type=jax.ShapeDtypeStruct((BATCH * MAXLEN * EMBED,), jnp.float32),
        mesh=plsc.VectorSubcoreMesh(core_axis_name="c", subcore_axis_name="s"),
        scratch_types=[
            pltpu.VMEM((T_PER_W * EMBED,), jnp.float32),
            pltpu.VMEM((WORDS_SUB,), jnp.float32),
            pltpu.VMEM((WORDS_SUB,), jnp.float32),
            pltpu.VMEM((WORDS_SUB,), jnp.float32),
            pltpu.SemaphoreType.DMA,
            pltpu.SemaphoreType.DMA,
            pltpu.SemaphoreType.DMA,
            pltpu.SemaphoreType.DMA,
            pltpu.SemaphoreType.DMA,
            pltpu.SemaphoreType.DMA,
        ],
    )(xf, pf)
    return out.reshape(BATCH, MAXLEN, EMBED)


SEQ_BLK = 512


def _add_body(x_ref, pos_ref, o_ref):
    o_ref[...] = x_ref[...] + pos_ref[...][None, :, :]


def _tc_add(x, pos_table):
    grid = (MAXLEN // SEQ_BLK,)
    return pl.pallas_call(
        _add_body,
        grid=grid,
        in_specs=[
            pl.BlockSpec((BATCH, SEQ_BLK, EMBED), lambda s: (0, s, 0)),
            pl.BlockSpec((SEQ_BLK, EMBED), lambda s: (s, 0)),
        ],
        out_specs=pl.BlockSpec((BATCH, SEQ_BLK, EMBED), lambda s: (0, s, 0)),
        out_shape=jax.ShapeDtypeStruct((BATCH, MAXLEN, EMBED), jnp.float32),
    )(x, pos_table)


def kernel(x, pos_table):
    return _sc_add(x, pos_table)


# speedup vs baseline: 2.4333x; 2.4333x over previous
import jax
import jax.numpy as jnp
from jax import lax
from jax.experimental import pallas as pl
from jax.experimental.pallas import tpu as pltpu
from jax.experimental.pallas import tpu_sc as plsc

BATCH, MAXLEN, EMBED = 4, 2048, 1024
NC, NS = 2, 16
NW = NC * NS
T_PER_W = MAXLEN // NW
ROWS_SUB = 16
WORDS_SUB = ROWS_SUB * EMBED
N_SUB = (T_PER_W // ROWS_SUB) * BATCH


def _sc_body(x_hbm, pos_hbm, out_hbm, pos_v, xb0, xb1, xb2,
             si0, si1, si2, so0, so1, so2):
    wid = lax.axis_index("s") * NC + lax.axis_index("c")
    t0 = wid * T_PER_W
    pltpu.sync_copy(pos_hbm.at[pl.ds(t0, T_PER_W), :], pos_v)

    bufs = (xb0, xb1, xb2)
    isems = (si0, si1, si2)
    osems = (so0, so1, so2)
    n_rc = T_PER_W // ROWS_SUB

    def start_in(step):
        p = step % 3
        b, rc = divmod(step, n_rc)
        return pltpu.async_copy(
            x_hbm.at[b, pl.ds(t0 + rc * ROWS_SUB, ROWS_SUB), :],
            bufs[p], isems[p])

    def start_out(step):
        p = step % 3
        b, rc = divmod(step, n_rc)
        return pltpu.async_copy(
            bufs[p],
            out_hbm.at[b, pl.ds(t0 + rc * ROWS_SUB, ROWS_SUB), :], osems[p])

    h_in = [start_in(0), start_in(1), start_in(2)]
    h_out = [None, None, None]
    for step in range(N_SUB):
        p = step % 3
        nxt = step + 1
        if 3 <= nxt < N_SUB:
            q = nxt % 3
            h_out[q].wait()
            h_in[q] = start_in(nxt)
        h_in[p].wait()
        buf = bufs[p]
        rc = step % n_rc

        @plsc.parallel_loop(0, ROWS_SUB * EMBED, 16, unroll=8)
        def _(i, _buf=buf, _pr0=rc * ROWS_SUB):
            r = i >> 10
            c = pl.multiple_of(i & (EMBED - 1), 16)
            _buf[r, pl.ds(c, 16)] = (
                _buf[r, pl.ds(c, 16)] + pos_v[_pr0 + r, pl.ds(c, 16)])

        h_out[p] = start_out(step)
    for p in range(3):
        h_out[p].wait()


def _sc_add(x, pos_table):
    return pl.kernel(
        _sc_body,
        out_type=jax.ShapeDtypeStruct((BATCH, MAXLEN, EMBED), jnp.float32),
        mesh=plsc.VectorSubcoreMesh(core_axis_name="c", subcore_axis_name="s"),
        scratch_types=[
            pltpu.VMEM((T_PER_W, EMBED), jnp.float32),
            pltpu.VMEM((ROWS_SUB, EMBED), jnp.float32),
            pltpu.VMEM((ROWS_SUB, EMBED), jnp.float32),
            pltpu.VMEM((ROWS_SUB, EMBED), jnp.float32),
            pltpu.SemaphoreType.DMA,
            pltpu.SemaphoreType.DMA,
            pltpu.SemaphoreType.DMA,
            pltpu.SemaphoreType.DMA,
            pltpu.SemaphoreType.DMA,
            pltpu.SemaphoreType.DMA,
        ],
    )(x, pos_table)


SEQ_BLK = 512


def _add_body(x_ref, pos_ref, o_ref):
    o_ref[...] = x_ref[...] + pos_ref[...][None, :, :]


def _tc_add(x, pos_table):
    grid = (MAXLEN // SEQ_BLK,)
    return pl.pallas_call(
        _add_body,
        grid=grid,
        in_specs=[
            pl.BlockSpec((BATCH, SEQ_BLK, EMBED), lambda s: (0, s, 0)),
            pl.BlockSpec((SEQ_BLK, EMBED), lambda s: (s, 0)),
        ],
        out_specs=pl.BlockSpec((BATCH, SEQ_BLK, EMBED), lambda s: (0, s, 0)),
        out_shape=jax.ShapeDtypeStruct((BATCH, MAXLEN, EMBED), jnp.float32),
    )(x, pos_table)


def kernel(x, pos_table):
    return _sc_add(x, pos_table)


# speedup vs baseline: 2.6397x; 1.0848x over previous
import jax
import jax.numpy as jnp
from jax import lax
from jax.experimental import pallas as pl
from jax.experimental.pallas import tpu as pltpu
from jax.experimental.pallas import tpu_sc as plsc

BATCH, MAXLEN, EMBED = 4, 2048, 1024
NC, NS = 2, 16
NW = NC * NS

T_SC = 512
T_PER_W = T_SC // NW
ROWS_SUB = min(16, T_PER_W)
WORDS_SUB = ROWS_SUB * EMBED
N_SUB = (T_PER_W // ROWS_SUB) * BATCH


def _sc_body(x_hbm, pos_hbm, out_hbm, pos_v, xb0, xb1, xb2,
             psem, si0, si1, si2, so0, so1, so2):
    wid = lax.axis_index("s") * NC + lax.axis_index("c")
    t0 = wid * T_PER_W
    h_pos = pltpu.async_copy(pos_hbm.at[pl.ds(t0, T_PER_W), :], pos_v, psem)

    bufs = (xb0, xb1, xb2)
    isems = (si0, si1, si2)
    osems = (so0, so1, so2)
    n_rc = T_PER_W // ROWS_SUB

    def start_in(step):
        p = step % 3
        b, rc = divmod(step, n_rc)
        return pltpu.async_copy(
            x_hbm.at[b, pl.ds(t0 + rc * ROWS_SUB, ROWS_SUB), :],
            bufs[p], isems[p])

    def start_out(step):
        p = step % 3
        b, rc = divmod(step, n_rc)
        return pltpu.async_copy(
            bufs[p],
            out_hbm.at[b, pl.ds(t0 + rc * ROWS_SUB, ROWS_SUB), :], osems[p])

    h_in = [start_in(s) for s in range(min(3, N_SUB))]
    h_out = [None, None, None]
    h_pos.wait()
    for step in range(N_SUB):
        p = step % 3
        nxt = step + 1
        if 3 <= nxt < N_SUB:
            q = nxt % 3
            h_out[q].wait()
            h_in[q] = start_in(nxt)
        h_in[p].wait()
        buf = bufs[p]
        rc = step % n_rc

        @plsc.parallel_loop(0, ROWS_SUB * EMBED, 16, unroll=8)
        def _(i, _buf=buf, _pr0=rc * ROWS_SUB):
            r = i >> 10
            c = pl.multiple_of(i & (EMBED - 1), 16)
            _buf[r, pl.ds(c, 16)] = (
                _buf[r, pl.ds(c, 16)] + pos_v[_pr0 + r, pl.ds(c, 16)])

        h_out[p] = start_out(step)
    for p in range(min(3, N_SUB)):
        h_out[p].wait()


def _sc_add(x, pos_table):
    return pl.kernel(
        _sc_body,
        out_type=jax.ShapeDtypeStruct((BATCH, T_SC, EMBED), jnp.float32),
        mesh=plsc.VectorSubcoreMesh(core_axis_name="c", subcore_axis_name="s"),
        scratch_types=[
            pltpu.VMEM((T_PER_W, EMBED), jnp.float32),
            pltpu.VMEM((ROWS_SUB, EMBED), jnp.float32),
            pltpu.VMEM((ROWS_SUB, EMBED), jnp.float32),
            pltpu.VMEM((ROWS_SUB, EMBED), jnp.float32),
            pltpu.SemaphoreType.DMA,
            pltpu.SemaphoreType.DMA,
            pltpu.SemaphoreType.DMA,
            pltpu.SemaphoreType.DMA,
            pltpu.SemaphoreType.DMA,
            pltpu.SemaphoreType.DMA,
            pltpu.SemaphoreType.DMA,
        ],
    )(x, pos_table)


SEQ_BLK = 512
SC_BLKS = T_SC // SEQ_BLK


def _add_body(x_ref, pos_ref, o_ref):
    o_ref[...] = x_ref[...] + pos_ref[...][None, :, :]


def _tc_add_tail(x, pos_table):
    grid = (MAXLEN // SEQ_BLK - SC_BLKS,)
    return pl.pallas_call(
        _add_body,
        grid=grid,
        in_specs=[
            pl.BlockSpec((BATCH, SEQ_BLK, EMBED), lambda s: (0, s + SC_BLKS, 0)),
            pl.BlockSpec((SEQ_BLK, EMBED), lambda s: (s + SC_BLKS, 0)),
        ],
        out_specs=pl.BlockSpec(
            (BATCH, SEQ_BLK, EMBED), lambda s: (0, s + SC_BLKS, 0)),
        out_shape=jax.ShapeDtypeStruct((BATCH, MAXLEN, EMBED), jnp.float32),
    )(x, pos_table)


def kernel(x, pos_table):
    sc_piece = _sc_add(x, pos_table)
    tc_full = _tc_add_tail(x, pos_table)
    return lax.dynamic_update_slice(tc_full, sc_piece, (0, 0, 0))


# speedup vs baseline: 2.7626x; 1.0466x over previous
import jax
import jax.numpy as jnp
from jax import lax
from jax.experimental import pallas as pl
from jax.experimental.pallas import tpu as pltpu
from jax.experimental.pallas import tpu_sc as plsc

BATCH, MAXLEN, EMBED = 4, 2048, 1024
NC, NS = 2, 16
NW = NC * NS

T_SC = 256
T_PER_W = T_SC // NW
ROWS_SUB = min(16, T_PER_W)
N_SUB = (T_PER_W // ROWS_SUB) * BATCH

SEQ_BLK = 256
SC_BLKS = T_SC // SEQ_BLK
TC_BLKS = MAXLEN // SEQ_BLK - SC_BLKS


def _sc_body(x_hbm, pos_hbm, out_hbm):
    def scoped(pos_v, xb0, xb1, xb2, psem, si0, si1, si2, so0, so1, so2):
        wid = lax.axis_index("s") * NC + lax.axis_index("c")
        t0 = wid * T_PER_W
        h_pos = pltpu.async_copy(pos_hbm.at[pl.ds(t0, T_PER_W), :], pos_v, psem)

        bufs = (xb0, xb1, xb2)
        isems = (si0, si1, si2)
        osems = (so0, so1, so2)
        n_rc = T_PER_W // ROWS_SUB

        def start_in(step):
            p = step % 3
            b, rc = divmod(step, n_rc)
            return pltpu.async_copy(
                x_hbm.at[b, pl.ds(t0 + rc * ROWS_SUB, ROWS_SUB), :],
                bufs[p], isems[p])

        def start_out(step):
            p = step % 3
            b, rc = divmod(step, n_rc)
            return pltpu.async_copy(
                bufs[p],
                out_hbm.at[b, pl.ds(t0 + rc * ROWS_SUB, ROWS_SUB), :],
                osems[p])

        h_in = [start_in(s) for s in range(min(3, N_SUB))]
        h_out = [None, None, None]
        h_pos.wait()
        for step in range(N_SUB):
            p = step % 3
            nxt = step + 1
            if 3 <= nxt < N_SUB:
                q = nxt % 3
                h_out[q].wait()
                h_in[q] = start_in(nxt)
            h_in[p].wait()
            buf = bufs[p]
            rc = step % n_rc

            @plsc.parallel_loop(0, ROWS_SUB * EMBED, 16, unroll=8)
            def _(i, _buf=buf, _pr0=rc * ROWS_SUB):
                r = i >> 10
                c = pl.multiple_of(i & (EMBED - 1), 16)
                _buf[r, pl.ds(c, 16)] = (
                    _buf[r, pl.ds(c, 16)] + pos_v[_pr0 + r, pl.ds(c, 16)])

            h_out[p] = start_out(step)
        for p in range(min(3, N_SUB)):
            h_out[p].wait()

    pl.run_scoped(
        scoped,
        pltpu.VMEM((T_PER_W, EMBED), jnp.float32),
        pltpu.VMEM((ROWS_SUB, EMBED), jnp.float32),
        pltpu.VMEM((ROWS_SUB, EMBED), jnp.float32),
        pltpu.VMEM((ROWS_SUB, EMBED), jnp.float32),
        pltpu.SemaphoreType.DMA,
        pltpu.SemaphoreType.DMA,
        pltpu.SemaphoreType.DMA,
        pltpu.SemaphoreType.DMA,
        pltpu.SemaphoreType.DMA,
        pltpu.SemaphoreType.DMA,
        pltpu.SemaphoreType.DMA,
    )


def _tc_inner(x_ref, pos_ref, o_ref):
    o_ref[...] = x_ref[...] + pos_ref[...][None, :, :]


def _sc_head(x, pos_table):
    return pl.kernel(
        _sc_body,
        out_type=jax.ShapeDtypeStruct((BATCH, T_SC, EMBED), jnp.float32),
        mesh=plsc.VectorSubcoreMesh(core_axis_name="c", subcore_axis_name="s"),
    )(x, pos_table)


def _tc_tail_call(x, pos_table):
    return pl.pallas_call(
        _tc_inner,
        grid=(TC_BLKS,),
        in_specs=[
            pl.BlockSpec((BATCH, SEQ_BLK, EMBED), lambda s: (0, s + SC_BLKS, 0)),
            pl.BlockSpec((SEQ_BLK, EMBED), lambda s: (s + SC_BLKS, 0)),
        ],
        out_specs=pl.BlockSpec(
            (BATCH, SEQ_BLK, EMBED), lambda s: (0, s + SC_BLKS, 0)),
        out_shape=jax.ShapeDtypeStruct((BATCH, MAXLEN, EMBED), jnp.float32),
    )(x, pos_table)


def kernel(x, pos_table):
    sc_piece = _sc_head(x, pos_table)
    tc_full = _tc_tail_call(x, pos_table)
    return lax.dynamic_update_slice(tc_full, sc_piece, (0, 0, 0))
